# baseline (speedup 1.0000x reference)
import jax
import jax.numpy as jnp
from jax import lax
from jax.experimental import pallas as pl
from jax.experimental.pallas import tpu as pltpu
from jax.experimental.pallas import tpu_sc as plsc

_B, _N, _C, _M = 8, 16384, 16, 128
_S = 512
_F = 3 + _C
_NW = 32
_NBOX = (_B * _M) // _NW
_L = 16
_IDXCAP = _S + _L


def _sc_body(pts_hbm, feats_hbm, boxes_hbm,
             pooled_hbm, empty_hbm,
             pts3_v, px_v, py_v, pz_v, boxbuf, idxbuf, fidx, rows_v,
             flat19, emptybuf, sem):
    wid = lax.axis_index("s") * 2 + lax.axis_index("c")
    gbase = wid * _NBOX
    b = gbase // _M
    pltpu.sync_copy(pts_hbm.at[pl.ds(b * 3 * _N, 3 * _N)], pts3_v)
    pltpu.sync_copy(boxes_hbm.at[pl.ds(gbase * 8, _NBOX * 8)], boxbuf)
    lanes = lax.iota(jnp.int32, _L)
    lanes3 = 3 * lanes
    lanes19 = 19 * lanes
    lane0 = lanes == 0
    badd = b * _N
    zerov = jnp.zeros((_L,), jnp.float32)

    def deint(base):
        i3 = jnp.full((_L,), 3 * base, jnp.int32) + lanes3
        px_v[pl.ds(base, _L)] = plsc.load_gather(pts3_v, [i3])
        py_v[pl.ds(base, _L)] = plsc.load_gather(pts3_v, [i3 + 1])
        pz_v[pl.ds(base, _L)] = plsc.load_gather(pts3_v, [i3 + 2])

    plsc.parallel_loop(0, _N, step=_L, unroll=8)(deint)

    def box_body(i, _):
        def param(p):
            return plsc.load_gather(
                boxbuf, [jnp.full((_L,), i * 8 + p, jnp.int32)])

        cx, cy, czc, hx, hy, hz, cosa, sina = [param(p) for p in range(8)]

        def chunk(base, ptr):
            pxv = px_v[pl.ds(base, _L)]
            pyv = py_v[pl.ds(base, _L)]
            pzv = pz_v[pl.ds(base, _L)]
            dxp = pxv - cx
            dyp = pyv - cy
            lx = dxp * cosa + dyp * sina
            ly = dyp * cosa - dxp * sina
            m = ((jnp.abs(pzv - czc) <= hz)
                 & (jnp.abs(lx) <= hx)
                 & (jnp.abs(ly) <= hy))
            plsc.store_compressed(
                idxbuf.at[pl.ds(ptr, _L)], base + lanes, mask=m)
            return jnp.minimum(
                ptr + plsc.all_reduce_population_count(m)[0], _S)

        cnt = plsc.parallel_loop(
            0, _N, step=_L, unroll=8, carry=jnp.int32(0))(chunk)
        cnt_v = jnp.full((_L,), cnt, jnp.int32)
        cnt_safe = jnp.maximum(cnt_v, 1)

        @pl.when(cnt > 0)
        def _nonempty():
            def build(j0):
                w = lax.rem(j0 + lanes, cnt_safe)
                src = plsc.load_gather(idxbuf, [w])
                fidx[pl.ds(j0, _L)] = src + badd
                o = jnp.full((_L,), 19 * j0, jnp.int32) + lanes19
                plsc.store_scatter(
                    flat19, [o], plsc.load_gather(px_v, [src]))
                plsc.store_scatter(
                    flat19, [o + 1], plsc.load_gather(py_v, [src]))
                plsc.store_scatter(
                    flat19, [o + 2], plsc.load_gather(pz_v, [src]))

            plsc.parallel_loop(0, _S, step=_L, unroll=4)(build)

            copies = [
                pltpu.async_copy(
                    feats_hbm.at[fidx.at[pl.ds(kk * 128, 128)]],
                    rows_v.at[pl.ds(kk * 128, 128)], sem)
                for kk in range(_S // 128)
            ]
            for cpy in copies:
                cpy.wait()

            def repack(j):
                plsc.store_scatter(
                    flat19, [jnp.full((_L,), 19 * j + 3, jnp.int32) + lanes],
                    rows_v[j])

            plsc.parallel_loop(0, _S, step=1, unroll=8)(repack)

        @pl.when(cnt == 0)
        def _empty():
            def zfill(o):
                flat19[pl.ds(o, _L)] = zerov

            plsc.parallel_loop(0, _S * _F, step=_L, unroll=8)(zfill)

        pltpu.sync_copy(
            flat19, pooled_hbm.at[pl.ds((gbase + i) * _S * _F, _S * _F)])
        plsc.store_scatter(emptybuf, [jnp.full((_L,), i, jnp.int32)],
                           (cnt_v == 0).astype(jnp.int32), mask=lane0)
        return 0

    lax.fori_loop(0, _NBOX, box_body, 0)
    pltpu.sync_copy(emptybuf, empty_hbm.at[pl.ds(gbase, _NBOX)])


def kernel(points, point_features, boxes3d):
    B, N, _ = points.shape
    M = boxes3d.shape[1]
    pts = points.reshape(B * N * 3)
    feats = point_features.reshape(B * N, _C)
    cx = boxes3d[:, :, 0]
    cy = boxes3d[:, :, 1]
    dz = boxes3d[:, :, 5]
    czc = boxes3d[:, :, 2] + dz / 2.0
    hx = boxes3d[:, :, 3] / 2.0
    hy = boxes3d[:, :, 4] / 2.0
    hz = dz / 2.0
    rz = boxes3d[:, :, 6]
    boxes_prep = jnp.stack(
        [cx, cy, czc, hx, hy, hz, jnp.cos(rz), jnp.sin(rz)],
        axis=-1).reshape(-1)

    mesh = plsc.VectorSubcoreMesh(core_axis_name="c", subcore_axis_name="s")
    sc = pl.kernel(
        _sc_body,
        out_type=(
            jax.ShapeDtypeStruct((B * M * _S * _F,), jnp.float32),
            jax.ShapeDtypeStruct((B * M,), jnp.int32),
        ),
        mesh=mesh,
        compiler_params=pltpu.CompilerParams(
            needs_layout_passes=False, use_tc_tiling_on_sc=False),
        scratch_types=[
            pltpu.VMEM((3 * _N,), jnp.float32),
            pltpu.VMEM((_N,), jnp.float32),
            pltpu.VMEM((_N,), jnp.float32),
            pltpu.VMEM((_N,), jnp.float32),
            pltpu.VMEM((_NBOX * 8,), jnp.float32),
            pltpu.VMEM((_IDXCAP,), jnp.int32),
            pltpu.VMEM((_S,), jnp.int32),
            pltpu.VMEM((_S, _C), jnp.float32),
            pltpu.VMEM((_S * _F,), jnp.float32),
            pltpu.VMEM((_NBOX,), jnp.int32),
            pltpu.SemaphoreType.DMA,
        ],
    )
    pooled_flat, empty_flat = sc(pts, feats, boxes_prep)
    return pooled_flat.reshape(B, M, _S, _F), empty_flat.reshape(B, M)


# speedup vs baseline: 1.1952x; 1.1952x over previous
import jax
import jax.numpy as jnp
from jax import lax
from jax.experimental import pallas as pl
from jax.experimental.pallas import tpu as pltpu
from jax.experimental.pallas import tpu_sc as plsc

_B, _N, _C, _M = 8, 16384, 16, 128
_S = 512
_F = 3 + _C
_NW = 32
_NBOX = (_B * _M) // _NW
_L = 16
_IDXCAP = _S + _L


def _sc_body(pts_hbm, feats_hbm, boxes_hbm,
             feat_out_hbm, xyz_out_hbm, empty_hbm,
             pts3_v, px_v, py_v, pz_v, boxbuf, idxbuf, fidx, rows_v,
             flat16, xyzbuf, emptybuf, sem):
    wid = lax.axis_index("s") * 2 + lax.axis_index("c")
    gbase = wid * _NBOX
    b = gbase // _M
    pltpu.sync_copy(pts_hbm.at[pl.ds(b * 3 * _N, 3 * _N)], pts3_v)
    pltpu.sync_copy(boxes_hbm.at[pl.ds(gbase * 8, _NBOX * 8)], boxbuf)
    lanes = lax.iota(jnp.int32, _L)
    lanes3 = 3 * lanes
    lane0 = lanes == 0
    badd = b * _N
    zerov = jnp.zeros((_L,), jnp.float32)

    def deint(base):
        i3 = jnp.full((_L,), 3 * base, jnp.int32) + lanes3
        px_v[pl.ds(base, _L)] = plsc.load_gather(pts3_v, [i3])
        py_v[pl.ds(base, _L)] = plsc.load_gather(pts3_v, [i3 + 1])
        pz_v[pl.ds(base, _L)] = plsc.load_gather(pts3_v, [i3 + 2])

    plsc.parallel_loop(0, _N, step=_L, unroll=8)(deint)

    def box_body(i, _):
        def param(p):
            return plsc.load_gather(
                boxbuf, [jnp.full((_L,), i * 8 + p, jnp.int32)])

        cx, cy, czc, hx, hy, hz, cosa, sina = [param(p) for p in range(8)]

        def chunk(base, ptr):
            pxv = px_v[pl.ds(base, _L)]
            pyv = py_v[pl.ds(base, _L)]
            pzv = pz_v[pl.ds(base, _L)]
            dxp = pxv - cx
            dyp = pyv - cy
            lx = dxp * cosa + dyp * sina
            ly = dyp * cosa - dxp * sina
            m = ((jnp.abs(pzv - czc) <= hz)
                 & (jnp.abs(lx) <= hx)
                 & (jnp.abs(ly) <= hy))
            plsc.store_compressed(
                idxbuf.at[pl.ds(ptr, _L)], base + lanes, mask=m)
            return jnp.minimum(
                ptr + plsc.all_reduce_population_count(m)[0], _S)

        cnt = plsc.parallel_loop(
            0, _N, step=_L, unroll=8, carry=jnp.int32(0))(chunk)
        cnt_v = jnp.full((_L,), cnt, jnp.int32)
        cnt_safe = jnp.maximum(cnt_v, 1)

        @pl.when(cnt > 0)
        def _nonempty():
            def build(j0):
                w = lax.rem(j0 + lanes, cnt_safe)
                src = plsc.load_gather(idxbuf, [w])
                fidx[pl.ds(j0, _L)] = src + badd
                xyzbuf[pl.ds(j0, _L)] = plsc.load_gather(px_v, [src])
                xyzbuf[pl.ds(_S + j0, _L)] = plsc.load_gather(py_v, [src])
                xyzbuf[pl.ds(2 * _S + j0, _L)] = plsc.load_gather(pz_v, [src])

            plsc.parallel_loop(0, _S, step=_L, unroll=4)(build)

            copies = [
                pltpu.async_copy(
                    feats_hbm.at[fidx.at[pl.ds(kk * 128, 128)]],
                    rows_v.at[pl.ds(kk * 128, 128)], sem)
                for kk in range(_S // 128)
            ]
            for cpy in copies:
                cpy.wait()

            def repack(j):
                flat16[pl.ds(_C * j, _L)] = rows_v[j]

            plsc.parallel_loop(0, _S, step=1, unroll=8)(repack)

        @pl.when(cnt == 0)
        def _empty():
            def zfill(o):
                flat16[pl.ds(o, _L)] = zerov

            plsc.parallel_loop(0, _S * _C, step=_L, unroll=8)(zfill)

            def zfill3(o):
                xyzbuf[pl.ds(o, _L)] = zerov

            plsc.parallel_loop(0, 3 * _S, step=_L, unroll=8)(zfill3)

        pltpu.sync_copy(
            flat16, feat_out_hbm.at[pl.ds((gbase + i) * _S * _C, _S * _C)])
        pltpu.sync_copy(
            xyzbuf, xyz_out_hbm.at[pl.ds((gbase + i) * 3 * _S, 3 * _S)])
        plsc.store_scatter(emptybuf, [jnp.full((_L,), i, jnp.int32)],
                           (cnt_v == 0).astype(jnp.int32), mask=lane0)
        return 0

    lax.fori_loop(0, _NBOX, box_body, 0)
    pltpu.sync_copy(emptybuf, empty_hbm.at[pl.ds(gbase, _NBOX)])


def kernel(points, point_features, boxes3d):
    B, N, _ = points.shape
    M = boxes3d.shape[1]
    pts = points.reshape(B * N * 3)
    feats = point_features.reshape(B * N, _C)
    cx = boxes3d[:, :, 0]
    cy = boxes3d[:, :, 1]
    dz = boxes3d[:, :, 5]
    czc = boxes3d[:, :, 2] + dz / 2.0
    hx = boxes3d[:, :, 3] / 2.0
    hy = boxes3d[:, :, 4] / 2.0
    hz = dz / 2.0
    rz = boxes3d[:, :, 6]
    boxes_prep = jnp.stack(
        [cx, cy, czc, hx, hy, hz, jnp.cos(rz), jnp.sin(rz)],
        axis=-1).reshape(-1)

    mesh = plsc.VectorSubcoreMesh(core_axis_name="c", subcore_axis_name="s")
    sc = pl.kernel(
        _sc_body,
        out_type=(
            jax.ShapeDtypeStruct((B * M * _S * _C,), jnp.float32),
            jax.ShapeDtypeStruct((B * M * 3 * _S,), jnp.float32),
            jax.ShapeDtypeStruct((B * M,), jnp.int32),
        ),
        mesh=mesh,
        compiler_params=pltpu.CompilerParams(
            needs_layout_passes=False, use_tc_tiling_on_sc=False),
        scratch_types=[
            pltpu.VMEM((3 * _N,), jnp.float32),
            pltpu.VMEM((_N,), jnp.float32),
            pltpu.VMEM((_N,), jnp.float32),
            pltpu.VMEM((_N,), jnp.float32),
            pltpu.VMEM((_NBOX * 8,), jnp.float32),
            pltpu.VMEM((_IDXCAP,), jnp.int32),
            pltpu.VMEM((_S,), jnp.int32),
            pltpu.VMEM((_S, _C), jnp.float32),
            pltpu.VMEM((_S * _C,), jnp.float32),
            pltpu.VMEM((3 * _S,), jnp.float32),
            pltpu.VMEM((_NBOX,), jnp.int32),
            pltpu.SemaphoreType.DMA,
        ],
    )
    feat_out, xyz_out, empty_flat = sc(pts, feats, boxes_prep)
    xyz = xyz_out.reshape(B, M, 3, _S).transpose(0, 1, 3, 2)
    pooled = jnp.concatenate([xyz, feat_out.reshape(B, M, _S, _C)], axis=-1)
    return pooled, empty_flat.reshape(B, M)


# speedup vs baseline: 1.2797x; 1.0707x over previous
import jax
import jax.numpy as jnp
from jax import lax
from jax.experimental import pallas as pl
from jax.experimental.pallas import tpu as pltpu
from jax.experimental.pallas import tpu_sc as plsc

_B, _N, _C, _M = 8, 16384, 16, 128
_S = 512
_F = 3 + _C
_NW = 32
_NBOX = (_B * _M) // _NW
_R = _B * _N
_ZROW = _R
_L = 16
_IDXCAP = _S + _L


def _sc_body(px_hbm, py_hbm, pz_hbm, feats_hbm, boxes_hbm,
             feat_out_hbm, xyz_out_hbm, empty_hbm,
             px_v, py_v, pz_v, boxbuf, idxbuf, fidx, rows_v, xyzbuf,
             emptybuf, sem):
    wid = lax.axis_index("s") * 2 + lax.axis_index("c")
    gbase = wid * _NBOX
    b = gbase // _M
    pltpu.sync_copy(px_hbm.at[pl.ds(b * _N, _N)], px_v)
    pltpu.sync_copy(py_hbm.at[pl.ds(b * _N, _N)], py_v)
    pltpu.sync_copy(pz_hbm.at[pl.ds(b * _N, _N)], pz_v)
    pltpu.sync_copy(boxes_hbm.at[pl.ds(gbase * 8, _NBOX * 8)], boxbuf)
    lanes = lax.iota(jnp.int32, _L)
    lane0 = lanes == 0
    badd = b * _N

    def box_body(i, _):
        def param(p):
            return plsc.load_gather(
                boxbuf, [jnp.full((_L,), i * 8 + p, jnp.int32)])

        cx, cy, czc, hx, hy, hz, cosa, sina = [param(p) for p in range(8)]

        def chunk(base, ptr):
            pxv = px_v[pl.ds(base, _L)]
            pyv = py_v[pl.ds(base, _L)]
            pzv = pz_v[pl.ds(base, _L)]
            dxp = pxv - cx
            dyp = pyv - cy
            lx = dxp * cosa + dyp * sina
            ly = dyp * cosa - dxp * sina
            m = ((jnp.abs(pzv - czc) <= hz)
                 & (jnp.abs(lx) <= hx)
                 & (jnp.abs(ly) <= hy))
            plsc.store_compressed(
                idxbuf.at[pl.ds(ptr, _L)], base + lanes, mask=m)
            return jnp.minimum(
                ptr + plsc.all_reduce_population_count(m)[0], _S)

        cnt = plsc.parallel_loop(
            0, _N, step=_L, unroll=8, carry=jnp.int32(0))(chunk)
        cnt_v = jnp.full((_L,), cnt, jnp.int32)
        empty_v = cnt_v == 0
        cnt_safe = jnp.maximum(cnt_v, 1)
        nzf = jnp.where(empty_v, 0.0, 1.0)

        def build(j0):
            w = lax.rem(j0 + lanes, cnt_safe)
            src = plsc.load_gather(idxbuf, [w])
            src = jnp.where(empty_v, 0, src)
            fidx[pl.ds(j0, _L)] = jnp.where(empty_v, _ZROW, src + badd)
            xyzbuf[pl.ds(j0, _L)] = plsc.load_gather(px_v, [src]) * nzf
            xyzbuf[pl.ds(_S + j0, _L)] = plsc.load_gather(py_v, [src]) * nzf
            xyzbuf[pl.ds(2 * _S + j0, _L)] = plsc.load_gather(pz_v, [src]) * nzf

        plsc.parallel_loop(0, _S, step=_L, unroll=4)(build)

        copies = [
            pltpu.async_copy(
                feats_hbm.at[fidx.at[pl.ds(kk * 128, 128)]],
                rows_v.at[pl.ds(kk * 128, 128)], sem)
            for kk in range(_S // 128)
        ]
        for cpy in copies:
            cpy.wait()
        pltpu.sync_copy(rows_v, feat_out_hbm.at[pl.ds((gbase + i) * _S, _S)])
        pltpu.sync_copy(
            xyzbuf, xyz_out_hbm.at[pl.ds((gbase + i) * 3 * _S, 3 * _S)])
        plsc.store_scatter(emptybuf, [jnp.full((_L,), i, jnp.int32)],
                           empty_v.astype(jnp.int32), mask=lane0)
        return 0

    lax.fori_loop(0, _NBOX, box_body, 0)
    pltpu.sync_copy(emptybuf, empty_hbm.at[pl.ds(gbase, _NBOX)])


def kernel(points, point_features, boxes3d):
    B, N, _ = points.shape
    M = boxes3d.shape[1]
    px = points[:, :, 0].reshape(-1)
    py = points[:, :, 1].reshape(-1)
    pz = points[:, :, 2].reshape(-1)
    feats = jnp.concatenate(
        [point_features.reshape(B * N, _C),
         jnp.zeros((_L, _C), jnp.float32)], axis=0)
    cx = boxes3d[:, :, 0]
    cy = boxes3d[:, :, 1]
    dz = boxes3d[:, :, 5]
    czc = boxes3d[:, :, 2] + dz / 2.0
    hx = boxes3d[:, :, 3] / 2.0
    hy = boxes3d[:, :, 4] / 2.0
    hz = dz / 2.0
    rz = boxes3d[:, :, 6]
    boxes_prep = jnp.stack(
        [cx, cy, czc, hx, hy, hz, jnp.cos(rz), jnp.sin(rz)],
        axis=-1).reshape(-1)

    mesh = plsc.VectorSubcoreMesh(core_axis_name="c", subcore_axis_name="s")
    sc = pl.kernel(
        _sc_body,
        out_type=(
            jax.ShapeDtypeStruct((B * M * _S, _C), jnp.float32),
            jax.ShapeDtypeStruct((B * M * 3 * _S,), jnp.float32),
            jax.ShapeDtypeStruct((B * M,), jnp.int32),
        ),
        mesh=mesh,
        compiler_params=pltpu.CompilerParams(
            needs_layout_passes=False, use_tc_tiling_on_sc=False),
        scratch_types=[
            pltpu.VMEM((_N,), jnp.float32),
            pltpu.VMEM((_N,), jnp.float32),
            pltpu.VMEM((_N,), jnp.float32),
            pltpu.VMEM((_NBOX * 8,), jnp.float32),
            pltpu.VMEM((_IDXCAP,), jnp.int32),
            pltpu.VMEM((_S,), jnp.int32),
            pltpu.VMEM((_S, _C), jnp.float32),
            pltpu.VMEM((3 * _S,), jnp.float32),
            pltpu.VMEM((_NBOX,), jnp.int32),
            pltpu.SemaphoreType.DMA,
        ],
    )
    feat_out, xyz_out, empty_flat = sc(px, py, pz, feats, boxes_prep)
    xyz = xyz_out.reshape(B, M, 3, _S).transpose(0, 1, 3, 2)
    pooled = jnp.concatenate([xyz, feat_out.reshape(B, M, _S, _C)], axis=-1)
    return pooled, empty_flat.reshape(B, M)
